# Initial kernel scaffold; baseline (speedup 1.0000x reference)
import jax
import jax.numpy as jnp
from jax.experimental import pallas as pl


def kernel(x, edge_index, edge_weight, W1, b1, W2, b2):
    raise NotImplementedError("write your pallas kernel here")


# speedup vs baseline: 12.9858x; 12.9858x over previous
import jax
import jax.numpy as jnp
from jax import lax
from jax.experimental import pallas as pl
from jax.experimental.pallas import tpu as pltpu
from jax.experimental.pallas import tpu_sc as plsc

NC = 2
NS = 16
NW = NC * NS
L = 16
CHUNK = 128


def _sc_edge_pass(n_nodes, ch, d, gather):
    mesh = plsc.VectorSubcoreMesh(core_axis_name="c", subcore_axis_name="s")
    rows_per_tile = n_nodes // NS
    n_full = rows_per_tile // CHUNK
    tail = rows_per_tile - n_full * CHUNK
    if not gather:
        assert d == L

    def body(*refs):
        if gather:
            t_hbm, row_hbm, col_hbm, ewx_hbm, out_hbm, rowv, colv, ew_buf, g_buf, acc, sem = refs
        else:
            col_hbm, ewx_hbm, out_hbm, colv, g_buf, acc, sem = refs
            t_hbm = row_hbm = rowv = ew_buf = None
        cid = lax.axis_index("c")
        sid = lax.axis_index("s")
        wid = sid * NC + cid

        if gather:
            pltpu.sync_copy(row_hbm.at[wid], rowv)
        pltpu.sync_copy(col_hbm.at[wid], colv)

        @pl.loop(0, CHUNK)
        def _(r):
            for f in range(d // L):
                g_buf[r, pl.ds(f * L, L)] = jnp.zeros((L,), jnp.float32)

        base = sid * rows_per_tile
        for k in range(n_full):
            pltpu.sync_copy(g_buf, acc.at[pl.ds(base + k * CHUNK, CHUNK)])
        if tail:
            pltpu.sync_copy(g_buf.at[pl.ds(0, tail)],
                            acc.at[pl.ds(base + n_full * CHUNK, tail)])
        plsc.subcore_barrier()

        @pl.loop(0, ch)
        def _(ci):
            if gather:
                pltpu.sync_copy(ewx_hbm.at[wid, ci], ew_buf)
                pltpu.async_copy(t_hbm.at[rowv.at[ci]], g_buf, sem).wait()

                @pl.loop(0, CHUNK)
                def _(r):
                    ew_s = ew_buf[r, :]
                    for f in range(d // L):
                        sl = pl.ds(f * L, L)
                        g_buf[r, sl] = g_buf[r, sl] * ew_s
            else:
                pltpu.sync_copy(ewx_hbm.at[wid, ci], g_buf)

            pltpu.sync_copy(g_buf, acc.at[colv.at[ci]], add=True)

        plsc.subcore_barrier()
        pltpu.sync_copy(acc.at[pl.ds(base, rows_per_tile)],
                        out_hbm.at[cid, pl.ds(base, rows_per_tile)])

    scratch = []
    if gather:
        scratch.append(pltpu.VMEM((ch, CHUNK), jnp.int32))
    scratch.append(pltpu.VMEM((ch, CHUNK), jnp.int32))
    if gather:
        scratch.append(pltpu.VMEM((CHUNK, L), jnp.float32))
    scratch += [
        pltpu.VMEM((CHUNK, d), jnp.float32),
        pltpu.VMEM_SHARED((n_nodes, d), jnp.float32),
        pltpu.SemaphoreType.DMA,
    ]
    return pl.kernel(
        body,
        out_type=jax.ShapeDtypeStruct((NC, n_nodes, d), jnp.float32),
        mesh=mesh,
        scratch_types=scratch,
        compiler_params=pltpu.CompilerParams(use_tc_tiling_on_sc=False),
    )


def _tc_prescale1(deg16, x, w1):
    n, fin = x.shape
    hid = w1.shape[1]
    bn = 1024

    def body(deg_ref, x_ref, w_ref, t1_ref, dinv_ref):
        deg = deg_ref[0, :, 0:1] + deg_ref[1, :, 0:1] + 1.0
        dinv = lax.rsqrt(deg)
        hw = jnp.dot(x_ref[...], w_ref[...], preferred_element_type=jnp.float32)
        t1_ref[...] = dinv * hw
        dinv_ref[...] = jnp.broadcast_to(dinv, (bn, hid))

    return pl.pallas_call(
        body,
        grid=(n // bn,),
        in_specs=[
            pl.BlockSpec((NC, bn, L), lambda i: (0, i, 0)),
            pl.BlockSpec((bn, fin), lambda i: (i, 0)),
            pl.BlockSpec((fin, hid), lambda i: (0, 0)),
        ],
        out_specs=[
            pl.BlockSpec((bn, hid), lambda i: (i, 0)),
            pl.BlockSpec((bn, hid), lambda i: (i, 0)),
        ],
        out_shape=[
            jax.ShapeDtypeStruct((n, hid), jnp.float32),
            jax.ShapeDtypeStruct((n, hid), jnp.float32),
        ],
    )(deg16, x, w1)


def _tc_mid(acc1, t1, dinv16, b1r, w2p):
    n, hid = t1.shape
    dp = w2p.shape[1]
    bn = 1024

    def body(acc_ref, t1_ref, dinv_ref, b1_ref, w2_ref, hid_ref, t2_ref):
        s = acc_ref[0] + acc_ref[1] + t1_ref[...]
        h = jnp.maximum(dinv_ref[...] * s + b1_ref[...], 0.0)
        hid_ref[...] = h
        hw2 = jnp.dot(h, w2_ref[...], preferred_element_type=jnp.float32)
        t2_ref[...] = dinv_ref[:, 0:1] * hw2

    return pl.pallas_call(
        body,
        grid=(n // bn,),
        in_specs=[
            pl.BlockSpec((NC, bn, hid), lambda i: (0, i, 0)),
            pl.BlockSpec((bn, hid), lambda i: (i, 0)),
            pl.BlockSpec((bn, hid), lambda i: (i, 0)),
            pl.BlockSpec((1, hid), lambda i: (0, 0)),
            pl.BlockSpec((hid, dp), lambda i: (0, 0)),
        ],
        out_specs=[
            pl.BlockSpec((bn, hid), lambda i: (i, 0)),
            pl.BlockSpec((bn, dp), lambda i: (i, 0)),
        ],
        out_shape=[
            jax.ShapeDtypeStruct((n, hid), jnp.float32),
            jax.ShapeDtypeStruct((n, dp), jnp.float32),
        ],
    )(acc1, t1, dinv16, b1r, w2p)


def _tc_post(acc2, t2, dinv16, b2r):
    n, dp = t2.shape
    c = b2r.shape[1]
    bn = 1024

    def body(acc_ref, t2_ref, dinv_ref, b2_ref, out_ref):
        s = acc_ref[0] + acc_ref[1] + t2_ref[...]
        o = dinv_ref[:, 0:1] * s
        out_ref[...] = o[:, :c] + b2_ref[...]

    return pl.pallas_call(
        body,
        grid=(n // bn,),
        in_specs=[
            pl.BlockSpec((NC, bn, dp), lambda i: (0, i, 0)),
            pl.BlockSpec((bn, dp), lambda i: (i, 0)),
            pl.BlockSpec((bn, L), lambda i: (i, 0)),
            pl.BlockSpec((1, c), lambda i: (0, 0)),
        ],
        out_specs=pl.BlockSpec((bn, c), lambda i: (i, 0)),
        out_shape=jax.ShapeDtypeStruct((n, c), jnp.float32),
    )(acc2, t2, dinv16, b2r)


def kernel(x, edge_index, edge_weight, W1, b1, W2, b2):
    n, _ = x.shape
    e = edge_weight.shape[0]
    hid = W1.shape[1]
    c = W2.shape[1]
    dp = ((c + L - 1) // L) * L
    npad = ((n + NS * CHUNK - 1) // (NS * CHUNK)) * (NS * CHUNK)

    epw = ((e + NW * CHUNK - 1) // (NW * CHUNK)) * CHUNK
    ch = epw // CHUNK
    epad = NW * epw - e
    row3 = jnp.pad(edge_index[0], (0, epad)).reshape(NW, ch, CHUNK)
    col3 = jnp.pad(edge_index[1], (0, epad)).reshape(NW, ch, CHUNK)
    ewx = jnp.broadcast_to(
        jnp.pad(edge_weight, (0, epad)).reshape(NW, ch, CHUNK, 1),
        (NW, ch, CHUNK, L))
    w2p = jnp.pad(W2, ((0, 0), (0, dp - c)))
    b1r = b1.reshape(1, hid)
    b2r = b2.reshape(1, c)
    xp = jnp.pad(x, ((0, npad - n), (0, 0)))

    deg16 = _sc_edge_pass(npad, ch, L, gather=False)(col3, ewx)
    t1, dinv16 = _tc_prescale1(deg16, xp, W1)
    acc1 = _sc_edge_pass(npad, ch, hid, gather=True)(t1, row3, col3, ewx)
    hidden, t2 = _tc_mid(acc1, t1, dinv16, b1r, w2p)
    acc2 = _sc_edge_pass(npad, ch, dp, gather=True)(t2, row3, col3, ewx)
    out = _tc_post(acc2, t2, dinv16, b2r)
    return (out[:n], hidden[:n])


# speedup vs baseline: 16.3987x; 1.2628x over previous
import jax
import jax.numpy as jnp
from jax import lax
from jax.experimental import pallas as pl
from jax.experimental.pallas import tpu as pltpu
from jax.experimental.pallas import tpu_sc as plsc

NC = 2
NS = 16
NW = NC * NS
L = 16
CHUNK = 128


def _sc_edge_pass(n_nodes, ch, d, gather):
    mesh = plsc.VectorSubcoreMesh(core_axis_name="c", subcore_axis_name="s")
    rows_per_tile = n_nodes // NS
    n_full = rows_per_tile // CHUNK
    tail = rows_per_tile - n_full * CHUNK
    if not gather:
        assert d == L

    assert ch % 2 == 0

    def body(*refs):
        if gather:
            (t_hbm, row_hbm, col_hbm, ewx_hbm, out_hbm,
             rowv, colv, ew_buf, g_buf, acc, sem0, sem1) = refs
        else:
            col_hbm, ewx_hbm, out_hbm, colv, g_buf, acc, sem0, sem1 = refs
            t_hbm = row_hbm = rowv = ew_buf = None
        sems = (sem0, sem1)
        cid = lax.axis_index("c")
        sid = lax.axis_index("s")
        wid = sid * NC + cid

        if gather:
            pltpu.sync_copy(row_hbm.at[wid], rowv)
        pltpu.sync_copy(col_hbm.at[wid], colv)

        @pl.loop(0, CHUNK)
        def _(r):
            for f in range(d // L):
                g_buf[0, r, pl.ds(f * L, L)] = jnp.zeros((L,), jnp.float32)

        base = sid * rows_per_tile
        for k in range(n_full):
            pltpu.sync_copy(g_buf.at[0], acc.at[pl.ds(base + k * CHUNK, CHUNK)])
        if tail:
            pltpu.sync_copy(g_buf.at[0, pl.ds(0, tail)],
                            acc.at[pl.ds(base + n_full * CHUNK, tail)])

        def issue(ci, b):
            if gather:
                pltpu.async_copy(ewx_hbm.at[wid, ci], ew_buf.at[b], sems[b])
                pltpu.async_copy(t_hbm.at[rowv.at[ci]], g_buf.at[b], sems[b])
            else:
                pltpu.async_copy(ewx_hbm.at[wid, ci], g_buf.at[b], sems[b])

        def drain(ci, b):
            if gather:
                pltpu.make_async_copy(
                    ewx_hbm.at[wid, ci], ew_buf.at[b], sems[b]).wait()
                pltpu.make_async_copy(
                    t_hbm.at[rowv.at[ci]], g_buf.at[b], sems[b]).wait()
            else:
                pltpu.make_async_copy(
                    ewx_hbm.at[wid, ci], g_buf.at[b], sems[b]).wait()

        issue(0, 0)
        plsc.subcore_barrier()

        @pl.loop(0, ch, step=2)
        def _(ci):
            for b in range(2):
                cur = ci + b
                nxt = cur + 1

                @pl.when(nxt < ch)
                def _():
                    issue(nxt, 1 - b)

                drain(cur, b)
                if gather:
                    @pl.loop(0, CHUNK)
                    def _(r):
                        ew_s = ew_buf[b, r, :]
                        for f in range(d // L):
                            sl = pl.ds(f * L, L)
                            g_buf[b, r, sl] = g_buf[b, r, sl] * ew_s
                pltpu.sync_copy(g_buf.at[b], acc.at[colv.at[cur]], add=True)

        plsc.subcore_barrier()
        pltpu.sync_copy(acc.at[pl.ds(base, rows_per_tile)],
                        out_hbm.at[cid, pl.ds(base, rows_per_tile)])

    scratch = []
    if gather:
        scratch.append(pltpu.VMEM((ch, CHUNK), jnp.int32))
    scratch.append(pltpu.VMEM((ch, CHUNK), jnp.int32))
    if gather:
        scratch.append(pltpu.VMEM((2, CHUNK, L), jnp.float32))
    scratch += [
        pltpu.VMEM((2, CHUNK, d), jnp.float32),
        pltpu.VMEM_SHARED((n_nodes, d), jnp.float32),
        pltpu.SemaphoreType.DMA,
        pltpu.SemaphoreType.DMA,
    ]
    return pl.kernel(
        body,
        out_type=jax.ShapeDtypeStruct((NC, n_nodes, d), jnp.float32),
        mesh=mesh,
        scratch_types=scratch,
        compiler_params=pltpu.CompilerParams(use_tc_tiling_on_sc=False),
    )


def _tc_prescale1(deg16, x, w1):
    n, fin = x.shape
    hid = w1.shape[1]
    bn = 1024

    def body(deg_ref, x_ref, w_ref, t1_ref, dinv_ref):
        deg = deg_ref[0, :, 0:1] + deg_ref[1, :, 0:1] + 1.0
        dinv = lax.rsqrt(deg)
        hw = jnp.dot(x_ref[...], w_ref[...], preferred_element_type=jnp.float32)
        t1_ref[...] = dinv * hw
        dinv_ref[...] = jnp.broadcast_to(dinv, (bn, hid))

    return pl.pallas_call(
        body,
        grid=(n // bn,),
        in_specs=[
            pl.BlockSpec((NC, bn, L), lambda i: (0, i, 0)),
            pl.BlockSpec((bn, fin), lambda i: (i, 0)),
            pl.BlockSpec((fin, hid), lambda i: (0, 0)),
        ],
        out_specs=[
            pl.BlockSpec((bn, hid), lambda i: (i, 0)),
            pl.BlockSpec((bn, hid), lambda i: (i, 0)),
        ],
        out_shape=[
            jax.ShapeDtypeStruct((n, hid), jnp.float32),
            jax.ShapeDtypeStruct((n, hid), jnp.float32),
        ],
    )(deg16, x, w1)


def _tc_mid(acc1, t1, dinv16, b1r, w2p):
    n, hid = t1.shape
    dp = w2p.shape[1]
    bn = 1024

    def body(acc_ref, t1_ref, dinv_ref, b1_ref, w2_ref, hid_ref, t2_ref):
        s = acc_ref[0] + acc_ref[1] + t1_ref[...]
        h = jnp.maximum(dinv_ref[...] * s + b1_ref[...], 0.0)
        hid_ref[...] = h
        hw2 = jnp.dot(h, w2_ref[...], preferred_element_type=jnp.float32)
        t2_ref[...] = dinv_ref[:, 0:1] * hw2

    return pl.pallas_call(
        body,
        grid=(n // bn,),
        in_specs=[
            pl.BlockSpec((NC, bn, hid), lambda i: (0, i, 0)),
            pl.BlockSpec((bn, hid), lambda i: (i, 0)),
            pl.BlockSpec((bn, hid), lambda i: (i, 0)),
            pl.BlockSpec((1, hid), lambda i: (0, 0)),
            pl.BlockSpec((hid, dp), lambda i: (0, 0)),
        ],
        out_specs=[
            pl.BlockSpec((bn, hid), lambda i: (i, 0)),
            pl.BlockSpec((bn, dp), lambda i: (i, 0)),
        ],
        out_shape=[
            jax.ShapeDtypeStruct((n, hid), jnp.float32),
            jax.ShapeDtypeStruct((n, dp), jnp.float32),
        ],
    )(acc1, t1, dinv16, b1r, w2p)


def _tc_post(acc2, t2, dinv16, b2r):
    n, dp = t2.shape
    c = b2r.shape[1]
    bn = 1024

    def body(acc_ref, t2_ref, dinv_ref, b2_ref, out_ref):
        s = acc_ref[0] + acc_ref[1] + t2_ref[...]
        o = dinv_ref[:, 0:1] * s
        out_ref[...] = o[:, :c] + b2_ref[...]

    return pl.pallas_call(
        body,
        grid=(n // bn,),
        in_specs=[
            pl.BlockSpec((NC, bn, dp), lambda i: (0, i, 0)),
            pl.BlockSpec((bn, dp), lambda i: (i, 0)),
            pl.BlockSpec((bn, L), lambda i: (i, 0)),
            pl.BlockSpec((1, c), lambda i: (0, 0)),
        ],
        out_specs=pl.BlockSpec((bn, c), lambda i: (i, 0)),
        out_shape=jax.ShapeDtypeStruct((n, c), jnp.float32),
    )(acc2, t2, dinv16, b2r)


def kernel(x, edge_index, edge_weight, W1, b1, W2, b2):
    n, _ = x.shape
    e = edge_weight.shape[0]
    hid = W1.shape[1]
    c = W2.shape[1]
    dp = ((c + L - 1) // L) * L
    npad = ((n + NS * CHUNK - 1) // (NS * CHUNK)) * (NS * CHUNK)

    epw = ((e + 2 * NW * CHUNK - 1) // (2 * NW * CHUNK)) * 2 * CHUNK
    ch = epw // CHUNK
    epad = NW * epw - e
    row3 = jnp.pad(edge_index[0], (0, epad)).reshape(NW, ch, CHUNK)
    col3 = jnp.pad(edge_index[1], (0, epad)).reshape(NW, ch, CHUNK)
    ewx = jnp.broadcast_to(
        jnp.pad(edge_weight, (0, epad)).reshape(NW, ch, CHUNK, 1),
        (NW, ch, CHUNK, L))
    w2p = jnp.pad(W2, ((0, 0), (0, dp - c)))
    b1r = b1.reshape(1, hid)
    b2r = b2.reshape(1, c)
    xp = jnp.pad(x, ((0, npad - n), (0, 0)))

    deg16 = _sc_edge_pass(npad, ch, L, gather=False)(col3, ewx)
    t1, dinv16 = _tc_prescale1(deg16, xp, W1)
    acc1 = _sc_edge_pass(npad, ch, hid, gather=True)(t1, row3, col3, ewx)
    hidden, t2 = _tc_mid(acc1, t1, dinv16, b1r, w2p)
    acc2 = _sc_edge_pass(npad, ch, dp, gather=True)(t2, row3, col3, ewx)
    out = _tc_post(acc2, t2, dinv16, b2r)
    return (out[:n], hidden[:n])


# speedup vs baseline: 21.5216x; 1.3124x over previous
import jax
import jax.numpy as jnp
from jax import lax
from jax.experimental import pallas as pl
from jax.experimental.pallas import tpu as pltpu
from jax.experimental.pallas import tpu_sc as plsc

NC = 2
NS = 16
NW = NC * NS
L = 16
CHUNK = 128


def _sc_edge_pass(n_nodes, ch, d, gather):
    mesh = plsc.VectorSubcoreMesh(core_axis_name="c", subcore_axis_name="s")
    rows_per_tile = n_nodes // NS
    n_full = rows_per_tile // CHUNK
    tail = rows_per_tile - n_full * CHUNK
    if not gather:
        assert d == L

    assert ch % 2 == 0

    def body(*refs):
        if gather:
            (t_hbm, row_hbm, col_hbm, ewx_hbm, out_hbm,
             rowv, colv, ew_buf, g_buf, acc, sem0, sem1) = refs
        else:
            col_hbm, ewx_hbm, out_hbm, colv, g_buf, acc, sem0, sem1 = refs
            t_hbm = row_hbm = rowv = ew_buf = None
        sems = (sem0, sem1)
        cid = lax.axis_index("c")
        sid = lax.axis_index("s")
        wid = sid * NC + cid

        if gather:
            pltpu.sync_copy(row_hbm.at[wid], rowv)
        pltpu.sync_copy(col_hbm.at[wid], colv)

        @pl.loop(0, CHUNK)
        def _(r):
            for f in range(d // L):
                g_buf[0, r, pl.ds(f * L, L)] = jnp.zeros((L,), jnp.float32)

        base = sid * rows_per_tile
        for k in range(n_full):
            pltpu.sync_copy(g_buf.at[0], acc.at[pl.ds(base + k * CHUNK, CHUNK)])
        if tail:
            pltpu.sync_copy(g_buf.at[0, pl.ds(0, tail)],
                            acc.at[pl.ds(base + n_full * CHUNK, tail)])

        def issue(ci, b):
            if gather:
                pltpu.async_copy(ewx_hbm.at[wid, ci], ew_buf.at[b], sems[b])
                pltpu.async_copy(t_hbm.at[rowv.at[ci]], g_buf.at[b], sems[b])
            else:
                pltpu.async_copy(ewx_hbm.at[wid, ci], g_buf.at[b], sems[b])

        def drain(ci, b):
            if gather:
                pltpu.make_async_copy(
                    ewx_hbm.at[wid, ci], ew_buf.at[b], sems[b]).wait()
                pltpu.make_async_copy(
                    t_hbm.at[rowv.at[ci]], g_buf.at[b], sems[b]).wait()
            else:
                pltpu.make_async_copy(
                    ewx_hbm.at[wid, ci], g_buf.at[b], sems[b]).wait()

        issue(0, 0)
        plsc.subcore_barrier()

        @pl.loop(0, ch, step=2)
        def _(ci):
            for b in range(2):
                cur = ci + b
                nxt = cur + 1

                @pl.when(nxt < ch)
                def _():
                    issue(nxt, 1 - b)

                drain(cur, b)
                if gather:
                    @pl.loop(0, CHUNK)
                    def _(r):
                        ew_s = ew_buf[b, r, :]
                        for f in range(d // L):
                            sl = pl.ds(f * L, L)
                            g_buf[b, r, sl] = g_buf[b, r, sl] * ew_s
                pltpu.sync_copy(g_buf.at[b], acc.at[colv.at[cur]], add=True)

        plsc.subcore_barrier()
        pltpu.sync_copy(acc.at[pl.ds(base, rows_per_tile)],
                        out_hbm.at[cid, pl.ds(base, rows_per_tile)])

    scratch = []
    if gather:
        scratch.append(pltpu.VMEM((ch, CHUNK), jnp.int32))
    scratch.append(pltpu.VMEM((ch, CHUNK), jnp.int32))
    if gather:
        scratch.append(pltpu.VMEM((2, CHUNK, L), jnp.float32))
    scratch += [
        pltpu.VMEM((2, CHUNK, d), jnp.float32),
        pltpu.VMEM_SHARED((n_nodes, d), jnp.float32),
        pltpu.SemaphoreType.DMA,
        pltpu.SemaphoreType.DMA,
    ]
    return pl.kernel(
        body,
        out_type=jax.ShapeDtypeStruct((NC, n_nodes, d), jnp.float32),
        mesh=mesh,
        scratch_types=scratch,
        compiler_params=pltpu.CompilerParams(use_tc_tiling_on_sc=False),
    )


def _tc_prescale1(deg16, x, w1):
    n, fin = x.shape
    hid = w1.shape[1]
    bn = 1024

    def body(deg_ref, x_ref, w_ref, t1_ref, dinv_ref):
        deg = deg_ref[0, :, 0:1] + deg_ref[1, :, 0:1] + 1.0
        dinv = lax.rsqrt(deg)
        hw = jnp.dot(x_ref[...], w_ref[...], preferred_element_type=jnp.float32)
        t1_ref[...] = dinv * hw
        dinv_ref[...] = jnp.broadcast_to(dinv, (bn, hid))

    return pl.pallas_call(
        body,
        grid=(n // bn,),
        in_specs=[
            pl.BlockSpec((NC, bn, L), lambda i: (0, i, 0)),
            pl.BlockSpec((bn, fin), lambda i: (i, 0)),
            pl.BlockSpec((fin, hid), lambda i: (0, 0)),
        ],
        out_specs=[
            pl.BlockSpec((bn, hid), lambda i: (i, 0)),
            pl.BlockSpec((bn, hid), lambda i: (i, 0)),
        ],
        out_shape=[
            jax.ShapeDtypeStruct((n, hid), jnp.float32),
            jax.ShapeDtypeStruct((n, hid), jnp.float32),
        ],
    )(deg16, x, w1)


def _tc_mid(acc1, t1, dinv16, b1r):
    n, hid = t1.shape
    bn = 1024

    def body(acc_ref, t1_ref, dinv_ref, b1_ref, hid_ref, h2_ref):
        s = acc_ref[0] + acc_ref[1] + t1_ref[...]
        h = jnp.maximum(dinv_ref[...] * s + b1_ref[...], 0.0)
        hid_ref[...] = h
        h2_ref[...] = dinv_ref[...] * h

    return pl.pallas_call(
        body,
        grid=(n // bn,),
        in_specs=[
            pl.BlockSpec((NC, bn, hid), lambda i: (0, i, 0)),
            pl.BlockSpec((bn, hid), lambda i: (i, 0)),
            pl.BlockSpec((bn, hid), lambda i: (i, 0)),
            pl.BlockSpec((1, hid), lambda i: (0, 0)),
        ],
        out_specs=[
            pl.BlockSpec((bn, hid), lambda i: (i, 0)),
            pl.BlockSpec((bn, hid), lambda i: (i, 0)),
        ],
        out_shape=[
            jax.ShapeDtypeStruct((n, hid), jnp.float32),
            jax.ShapeDtypeStruct((n, hid), jnp.float32),
        ],
    )(acc1, t1, dinv16, b1r)


def _tc_post(acc2, h2, dinv16, w2, b2r):
    n, hid = h2.shape
    c = b2r.shape[1]
    bn = 1024

    def body(acc_ref, h2_ref, dinv_ref, w2_ref, b2_ref, out_ref):
        s = acc_ref[0] + acc_ref[1] + h2_ref[...]
        o = dinv_ref[...] * s
        out_ref[...] = jnp.dot(
            o, w2_ref[...], preferred_element_type=jnp.float32) + b2_ref[...]

    return pl.pallas_call(
        body,
        grid=(n // bn,),
        in_specs=[
            pl.BlockSpec((NC, bn, hid), lambda i: (0, i, 0)),
            pl.BlockSpec((bn, hid), lambda i: (i, 0)),
            pl.BlockSpec((bn, hid), lambda i: (i, 0)),
            pl.BlockSpec((hid, c), lambda i: (0, 0)),
            pl.BlockSpec((1, c), lambda i: (0, 0)),
        ],
        out_specs=pl.BlockSpec((bn, c), lambda i: (i, 0)),
        out_shape=jax.ShapeDtypeStruct((n, c), jnp.float32),
    )(acc2, h2, dinv16, w2, b2r)


def kernel(x, edge_index, edge_weight, W1, b1, W2, b2):
    n, _ = x.shape
    e = edge_weight.shape[0]
    hid = W1.shape[1]
    npad = ((n + NS * CHUNK - 1) // (NS * CHUNK)) * (NS * CHUNK)

    epw = ((e + 2 * NW * CHUNK - 1) // (2 * NW * CHUNK)) * 2 * CHUNK
    ch = epw // CHUNK
    epad = NW * epw - e
    row3 = jnp.pad(edge_index[0], (0, epad)).reshape(NW, ch, CHUNK)
    col3 = jnp.pad(edge_index[1], (0, epad)).reshape(NW, ch, CHUNK)
    ewx = jnp.broadcast_to(
        jnp.pad(edge_weight, (0, epad)).reshape(NW, ch, CHUNK, 1),
        (NW, ch, CHUNK, L))
    b1r = b1.reshape(1, hid)
    b2r = b2.reshape(1, W2.shape[1])
    xp = jnp.pad(x, ((0, npad - n), (0, 0)))

    deg16 = _sc_edge_pass(npad, ch, L, gather=False)(col3, ewx)
    t1, dinv16 = _tc_prescale1(deg16, xp, W1)
    acc1 = _sc_edge_pass(npad, ch, hid, gather=True)(t1, row3, col3, ewx)
    hidden, h2 = _tc_mid(acc1, t1, dinv16, b1r)
    acc2 = _sc_edge_pass(npad, ch, hid, gather=True)(h2, row3, col3, ewx)
    out = _tc_post(acc2, h2, dinv16, W2, b2r)
    return (out[:n], hidden[:n])


# speedup vs baseline: 35.6068x; 1.6545x over previous
import jax
import jax.numpy as jnp
from jax import lax
from jax.experimental import pallas as pl
from jax.experimental.pallas import tpu as pltpu
from jax.experimental.pallas import tpu_sc as plsc

NC = 2
NS = 16
NW = NC * NS
L = 16
CHUNK = 128


def _sc_edge_pass(n_nodes, ch, d, gather):
    mesh = plsc.VectorSubcoreMesh(core_axis_name="c", subcore_axis_name="s")
    rows_per_tile = n_nodes // NS
    n_full = rows_per_tile // CHUNK
    tail = rows_per_tile - n_full * CHUNK
    if not gather:
        assert d == L

    assert ch % 2 == 0

    def body(*refs):
        if gather:
            (t_hbm, row_hbm, col_hbm, ew_hbm, out_hbm,
             rowv, colv, ewv, g_buf, acc, sem0, sem1) = refs
        else:
            col_hbm, ew_hbm, out_hbm, colv, ewv, g_buf, acc, sem0, sem1 = refs
            t_hbm = row_hbm = rowv = None
        sems = (sem0, sem1)
        cid = lax.axis_index("c")
        sid = lax.axis_index("s")
        wid = sid * NC + cid

        if gather:
            pltpu.sync_copy(row_hbm.at[wid], rowv)
        pltpu.sync_copy(col_hbm.at[wid], colv)
        pltpu.sync_copy(ew_hbm.at[wid], ewv)

        idxs = [jnp.full((L,), j, jnp.int32) for j in range(L)]

        @pl.loop(0, CHUNK)
        def _(r):
            for f in range(d // L):
                g_buf[0, r, pl.ds(f * L, L)] = jnp.zeros((L,), jnp.float32)

        base = sid * rows_per_tile
        for k in range(n_full):
            pltpu.sync_copy(g_buf.at[0], acc.at[pl.ds(base + k * CHUNK, CHUNK)])
        if tail:
            pltpu.sync_copy(g_buf.at[0, pl.ds(0, tail)],
                            acc.at[pl.ds(base + n_full * CHUNK, tail)])

        def issue(ci, b):
            pltpu.async_copy(t_hbm.at[rowv.at[ci]], g_buf.at[b], sems[b])

        def drain(ci, b):
            pltpu.make_async_copy(
                t_hbm.at[rowv.at[ci]], g_buf.at[b], sems[b]).wait()

        def scale(ci, b):
            @pl.loop(0, CHUNK // L)
            def _(g):
                ew16 = ewv[ci, pl.ds(g * L, L)]
                for j in range(L):
                    r = g * L + j
                    s = ew16.at[idxs[j]].get(mode="promise_in_bounds")
                    if gather:
                        for f in range(d // L):
                            sl = pl.ds(f * L, L)
                            g_buf[b, r, sl] = g_buf[b, r, sl] * s
                    else:
                        sl0 = pl.ds(0, L)
                        g_buf[b, r, sl0] = g_buf[b, r, sl0] * 0.0 + s

        if gather:
            issue(0, 0)
        plsc.subcore_barrier()

        @pl.loop(0, ch, step=2)
        def _(ci):
            for b in range(2):
                cur = ci + b
                nxt = cur + 1

                if gather:
                    @pl.when(nxt < ch)
                    def _():
                        issue(nxt, 1 - b)

                    drain(cur, b)
                scale(cur, b)
                pltpu.sync_copy(g_buf.at[b], acc.at[colv.at[cur]], add=True)

        plsc.subcore_barrier()
        pltpu.sync_copy(acc.at[pl.ds(base, rows_per_tile)],
                        out_hbm.at[cid, pl.ds(base, rows_per_tile)])

    scratch = []
    if gather:
        scratch.append(pltpu.VMEM((ch, CHUNK), jnp.int32))
    scratch += [
        pltpu.VMEM((ch, CHUNK), jnp.int32),
        pltpu.VMEM((ch, CHUNK), jnp.float32),
        pltpu.VMEM((2, CHUNK, d), jnp.float32),
        pltpu.VMEM_SHARED((n_nodes, d), jnp.float32),
        pltpu.SemaphoreType.DMA,
        pltpu.SemaphoreType.DMA,
    ]
    return pl.kernel(
        body,
        out_type=jax.ShapeDtypeStruct((NC, n_nodes, d), jnp.float32),
        mesh=mesh,
        scratch_types=scratch,
        compiler_params=pltpu.CompilerParams(use_tc_tiling_on_sc=False),
    )


def _tc_prescale1(deg16, x, w1):
    n, fin = x.shape
    hid = w1.shape[1]
    bn = 1024

    def body(deg_ref, x_ref, w_ref, t1_ref, dinv_ref):
        deg = deg_ref[0, :, 0:1] + deg_ref[1, :, 0:1] + 1.0
        dinv = lax.rsqrt(deg)
        hw = jnp.dot(x_ref[...], w_ref[...], preferred_element_type=jnp.float32)
        t1_ref[...] = dinv * hw
        dinv_ref[...] = jnp.broadcast_to(dinv, (bn, hid))

    return pl.pallas_call(
        body,
        grid=(n // bn,),
        in_specs=[
            pl.BlockSpec((NC, bn, L), lambda i: (0, i, 0)),
            pl.BlockSpec((bn, fin), lambda i: (i, 0)),
            pl.BlockSpec((fin, hid), lambda i: (0, 0)),
        ],
        out_specs=[
            pl.BlockSpec((bn, hid), lambda i: (i, 0)),
            pl.BlockSpec((bn, hid), lambda i: (i, 0)),
        ],
        out_shape=[
            jax.ShapeDtypeStruct((n, hid), jnp.float32),
            jax.ShapeDtypeStruct((n, hid), jnp.float32),
        ],
    )(deg16, x, w1)


def _tc_mid(acc1, t1, dinv16, b1r):
    n, hid = t1.shape
    bn = 1024

    def body(acc_ref, t1_ref, dinv_ref, b1_ref, hid_ref, h2_ref):
        s = acc_ref[0] + acc_ref[1] + t1_ref[...]
        h = jnp.maximum(dinv_ref[...] * s + b1_ref[...], 0.0)
        hid_ref[...] = h
        h2_ref[...] = dinv_ref[...] * h

    return pl.pallas_call(
        body,
        grid=(n // bn,),
        in_specs=[
            pl.BlockSpec((NC, bn, hid), lambda i: (0, i, 0)),
            pl.BlockSpec((bn, hid), lambda i: (i, 0)),
            pl.BlockSpec((bn, hid), lambda i: (i, 0)),
            pl.BlockSpec((1, hid), lambda i: (0, 0)),
        ],
        out_specs=[
            pl.BlockSpec((bn, hid), lambda i: (i, 0)),
            pl.BlockSpec((bn, hid), lambda i: (i, 0)),
        ],
        out_shape=[
            jax.ShapeDtypeStruct((n, hid), jnp.float32),
            jax.ShapeDtypeStruct((n, hid), jnp.float32),
        ],
    )(acc1, t1, dinv16, b1r)


def _tc_post(acc2, h2, dinv16, w2, b2r):
    n, hid = h2.shape
    c = b2r.shape[1]
    bn = 1024

    def body(acc_ref, h2_ref, dinv_ref, w2_ref, b2_ref, out_ref):
        s = acc_ref[0] + acc_ref[1] + h2_ref[...]
        o = dinv_ref[...] * s
        out_ref[...] = jnp.dot(
            o, w2_ref[...], preferred_element_type=jnp.float32) + b2_ref[...]

    return pl.pallas_call(
        body,
        grid=(n // bn,),
        in_specs=[
            pl.BlockSpec((NC, bn, hid), lambda i: (0, i, 0)),
            pl.BlockSpec((bn, hid), lambda i: (i, 0)),
            pl.BlockSpec((bn, hid), lambda i: (i, 0)),
            pl.BlockSpec((hid, c), lambda i: (0, 0)),
            pl.BlockSpec((1, c), lambda i: (0, 0)),
        ],
        out_specs=pl.BlockSpec((bn, c), lambda i: (i, 0)),
        out_shape=jax.ShapeDtypeStruct((n, c), jnp.float32),
    )(acc2, h2, dinv16, w2, b2r)


def kernel(x, edge_index, edge_weight, W1, b1, W2, b2):
    n, _ = x.shape
    e = edge_weight.shape[0]
    hid = W1.shape[1]
    npad = ((n + NS * CHUNK - 1) // (NS * CHUNK)) * (NS * CHUNK)

    epw = ((e + 2 * NW * CHUNK - 1) // (2 * NW * CHUNK)) * 2 * CHUNK
    ch = epw // CHUNK
    epad = NW * epw - e
    row3 = jnp.pad(edge_index[0], (0, epad)).reshape(NW, ch, CHUNK)
    col3 = jnp.pad(edge_index[1], (0, epad)).reshape(NW, ch, CHUNK)
    ew3 = jnp.pad(edge_weight, (0, epad)).reshape(NW, ch, CHUNK)
    b1r = b1.reshape(1, hid)
    b2r = b2.reshape(1, W2.shape[1])
    xp = jnp.pad(x, ((0, npad - n), (0, 0)))

    deg16 = _sc_edge_pass(npad, ch, L, gather=False)(col3, ew3)
    t1, dinv16 = _tc_prescale1(deg16, xp, W1)
    acc1 = _sc_edge_pass(npad, ch, hid, gather=True)(t1, row3, col3, ew3)
    hidden, h2 = _tc_mid(acc1, t1, dinv16, b1r)
    acc2 = _sc_edge_pass(npad, ch, hid, gather=True)(h2, row3, col3, ew3)
    out = _tc_post(acc2, h2, dinv16, W2, b2r)
    return (out[:n], hidden[:n])


# speedup vs baseline: 36.1831x; 1.0162x over previous
import jax
import jax.numpy as jnp
from jax import lax
from jax.experimental import pallas as pl
from jax.experimental.pallas import tpu as pltpu
from jax.experimental.pallas import tpu_sc as plsc

NC = 2
NS = 16
NW = NC * NS
L = 16
CHUNK = 128


def _sc_edge_pass(n_nodes, ch, d, gather):
    mesh = plsc.VectorSubcoreMesh(core_axis_name="c", subcore_axis_name="s")
    rows_per_tile = n_nodes // NS
    n_full = rows_per_tile // CHUNK
    tail = rows_per_tile - n_full * CHUNK
    if not gather:
        assert d == L

    assert ch % 2 == 0

    def body(*refs):
        if gather:
            (t_hbm, row_hbm, col_hbm, ew_hbm, out_hbm,
             rowv, colv, ewv, g_buf, acc, sem0, sem1, ssem0, ssem1) = refs
        else:
            (col_hbm, ew_hbm, out_hbm,
             colv, ewv, g_buf, acc, sem0, sem1, ssem0, ssem1) = refs
            t_hbm = row_hbm = rowv = None
        sems = (sem0, sem1)
        ssems = (ssem0, ssem1)
        cid = lax.axis_index("c")
        sid = lax.axis_index("s")
        wid = sid * NC + cid

        if gather:
            pltpu.sync_copy(row_hbm.at[wid], rowv)
        pltpu.sync_copy(col_hbm.at[wid], colv)
        pltpu.sync_copy(ew_hbm.at[wid], ewv)

        idxs = [jnp.full((L,), j, jnp.int32) for j in range(L)]

        @pl.loop(0, CHUNK)
        def _(r):
            for f in range(d // L):
                g_buf[0, r, pl.ds(f * L, L)] = jnp.zeros((L,), jnp.float32)

        base = sid * rows_per_tile
        for k in range(n_full):
            pltpu.sync_copy(g_buf.at[0], acc.at[pl.ds(base + k * CHUNK, CHUNK)])
        if tail:
            pltpu.sync_copy(g_buf.at[0, pl.ds(0, tail)],
                            acc.at[pl.ds(base + n_full * CHUNK, tail)])

        def issue(ci, b):
            pltpu.async_copy(t_hbm.at[rowv.at[ci]], g_buf.at[b], sems[b])

        def drain(ci, b):
            pltpu.make_async_copy(
                t_hbm.at[rowv.at[ci]], g_buf.at[b], sems[b]).wait()

        def scale(ci, b):
            for g in range(CHUNK // L):
                ew16 = ewv[ci, pl.ds(g * L, L)]
                for j in range(L):
                    r = g * L + j
                    s = ew16.at[idxs[j]].get(mode="promise_in_bounds")
                    if gather:
                        for f in range(d // L):
                            sl = pl.ds(f * L, L)
                            g_buf[b, r, sl] = g_buf[b, r, sl] * s
                    else:
                        sl0 = pl.ds(0, L)
                        g_buf[b, r, sl0] = g_buf[b, r, sl0] * 0.0 + s

        if gather:
            issue(0, 0)
        plsc.subcore_barrier()

        def scat_wait(ci, b):
            pltpu.make_async_copy(
                g_buf.at[b], acc.at[colv.at[ci]], ssems[b]).wait()

        @pl.loop(0, ch, step=2)
        def _(ci):
            for b in range(2):
                cur = ci + b
                nxt = cur + 1

                if gather:
                    @pl.when(nxt < ch)
                    def _():
                        @pl.when(nxt >= 2)
                        def _():
                            scat_wait(nxt - 2, 1 - b)
                        issue(nxt, 1 - b)

                    drain(cur, b)
                else:
                    @pl.when(cur >= 2)
                    def _():
                        scat_wait(cur - 2, b)
                scale(cur, b)
                pltpu.async_copy(g_buf.at[b], acc.at[colv.at[cur]],
                                 ssems[b], add=True)

        for b in range(2):
            scat_wait(ch - 2 + b, b)
        plsc.subcore_barrier()
        pltpu.sync_copy(acc.at[pl.ds(base, rows_per_tile)],
                        out_hbm.at[cid, pl.ds(base, rows_per_tile)])

    scratch = []
    if gather:
        scratch.append(pltpu.VMEM((ch, CHUNK), jnp.int32))
    scratch += [
        pltpu.VMEM((ch, CHUNK), jnp.int32),
        pltpu.VMEM((ch, CHUNK), jnp.float32),
        pltpu.VMEM((2, CHUNK, d), jnp.float32),
        pltpu.VMEM_SHARED((n_nodes, d), jnp.float32),
        pltpu.SemaphoreType.DMA,
        pltpu.SemaphoreType.DMA,
        pltpu.SemaphoreType.DMA,
        pltpu.SemaphoreType.DMA,
    ]
    return pl.kernel(
        body,
        out_type=jax.ShapeDtypeStruct((NC, n_nodes, d), jnp.float32),
        mesh=mesh,
        scratch_types=scratch,
        compiler_params=pltpu.CompilerParams(use_tc_tiling_on_sc=False),
    )


def _tc_matmul1(x, w1):
    n, fin = x.shape
    hid = w1.shape[1]
    bn = 1024

    def body(x_ref, w_ref, hw_ref):
        hw_ref[...] = jnp.dot(
            x_ref[...], w_ref[...], preferred_element_type=jnp.float32)

    return pl.pallas_call(
        body,
        grid=(n // bn,),
        in_specs=[
            pl.BlockSpec((bn, fin), lambda i: (i, 0)),
            pl.BlockSpec((fin, hid), lambda i: (0, 0)),
        ],
        out_specs=pl.BlockSpec((bn, hid), lambda i: (i, 0)),
        out_shape=jax.ShapeDtypeStruct((n, hid), jnp.float32),
    )(x, w1)


def _tc_prescale1(deg16, hw):
    n, hid = hw.shape
    bn = 1024

    def body(deg_ref, hw_ref, t1_ref, dinv_ref):
        deg = deg_ref[0, :, 0:1] + deg_ref[1, :, 0:1] + 1.0
        dinv = lax.rsqrt(deg)
        t1_ref[...] = dinv * hw_ref[...]
        dinv_ref[...] = jnp.broadcast_to(dinv, (bn, hid))

    return pl.pallas_call(
        body,
        grid=(n // bn,),
        in_specs=[
            pl.BlockSpec((NC, bn, L), lambda i: (0, i, 0)),
            pl.BlockSpec((bn, hid), lambda i: (i, 0)),
        ],
        out_specs=[
            pl.BlockSpec((bn, hid), lambda i: (i, 0)),
            pl.BlockSpec((bn, hid), lambda i: (i, 0)),
        ],
        out_shape=[
            jax.ShapeDtypeStruct((n, hid), jnp.float32),
            jax.ShapeDtypeStruct((n, hid), jnp.float32),
        ],
    )(deg16, hw)


def _tc_mid(acc1, t1, dinv16, b1r):
    n, hid = t1.shape
    bn = 1024

    def body(acc_ref, t1_ref, dinv_ref, b1_ref, hid_ref, h2_ref):
        s = acc_ref[0] + acc_ref[1] + t1_ref[...]
        h = jnp.maximum(dinv_ref[...] * s + b1_ref[...], 0.0)
        hid_ref[...] = h
        h2_ref[...] = dinv_ref[...] * h

    return pl.pallas_call(
        body,
        grid=(n // bn,),
        in_specs=[
            pl.BlockSpec((NC, bn, hid), lambda i: (0, i, 0)),
            pl.BlockSpec((bn, hid), lambda i: (i, 0)),
            pl.BlockSpec((bn, hid), lambda i: (i, 0)),
            pl.BlockSpec((1, hid), lambda i: (0, 0)),
        ],
        out_specs=[
            pl.BlockSpec((bn, hid), lambda i: (i, 0)),
            pl.BlockSpec((bn, hid), lambda i: (i, 0)),
        ],
        out_shape=[
            jax.ShapeDtypeStruct((n, hid), jnp.float32),
            jax.ShapeDtypeStruct((n, hid), jnp.float32),
        ],
    )(acc1, t1, dinv16, b1r)


def _tc_post(acc2, h2, dinv16, w2, b2r):
    n, hid = h2.shape
    c = b2r.shape[1]
    bn = 1024

    def body(acc_ref, h2_ref, dinv_ref, w2_ref, b2_ref, out_ref):
        s = acc_ref[0] + acc_ref[1] + h2_ref[...]
        o = dinv_ref[...] * s
        out_ref[...] = jnp.dot(
            o, w2_ref[...], preferred_element_type=jnp.float32) + b2_ref[...]

    return pl.pallas_call(
        body,
        grid=(n // bn,),
        in_specs=[
            pl.BlockSpec((NC, bn, hid), lambda i: (0, i, 0)),
            pl.BlockSpec((bn, hid), lambda i: (i, 0)),
            pl.BlockSpec((bn, hid), lambda i: (i, 0)),
            pl.BlockSpec((hid, c), lambda i: (0, 0)),
            pl.BlockSpec((1, c), lambda i: (0, 0)),
        ],
        out_specs=pl.BlockSpec((bn, c), lambda i: (i, 0)),
        out_shape=jax.ShapeDtypeStruct((n, c), jnp.float32),
    )(acc2, h2, dinv16, w2, b2r)


def kernel(x, edge_index, edge_weight, W1, b1, W2, b2):
    n, _ = x.shape
    e = edge_weight.shape[0]
    hid = W1.shape[1]
    npad = ((n + NS * CHUNK - 1) // (NS * CHUNK)) * (NS * CHUNK)

    epw = ((e + 2 * NW * CHUNK - 1) // (2 * NW * CHUNK)) * 2 * CHUNK
    ch = epw // CHUNK
    epad = NW * epw - e
    row3 = jnp.pad(edge_index[0], (0, epad)).reshape(NW, ch, CHUNK)
    col3 = jnp.pad(edge_index[1], (0, epad)).reshape(NW, ch, CHUNK)
    ew3 = jnp.pad(edge_weight, (0, epad)).reshape(NW, ch, CHUNK)
    b1r = b1.reshape(1, hid)
    b2r = b2.reshape(1, W2.shape[1])
    xp = jnp.pad(x, ((0, npad - n), (0, 0)))

    hw = _tc_matmul1(xp, W1)
    deg16 = _sc_edge_pass(npad, ch, L, gather=False)(col3, ew3)
    t1, dinv16 = _tc_prescale1(deg16, hw)
    acc1 = _sc_edge_pass(npad, ch, hid, gather=True)(t1, row3, col3, ew3)
    hidden, h2 = _tc_mid(acc1, t1, dinv16, b1r)
    acc2 = _sc_edge_pass(npad, ch, hid, gather=True)(h2, row3, col3, ew3)
    out = _tc_post(acc2, h2, dinv16, W2, b2r)
    return (out[:n], hidden[:n])


# speedup vs baseline: 36.2607x; 1.0021x over previous
import jax
import jax.numpy as jnp
from jax import lax
from jax.experimental import pallas as pl
from jax.experimental.pallas import tpu as pltpu
from jax.experimental.pallas import tpu_sc as plsc

NC = 2
NS = 16
NW = NC * NS
L = 16
CHUNK = 128


def _sc_edge_pass(n_nodes, ch, d, gather):
    mesh = plsc.VectorSubcoreMesh(core_axis_name="c", subcore_axis_name="s")
    rows_per_tile = n_nodes // NS
    n_full = rows_per_tile // CHUNK
    tail = rows_per_tile - n_full * CHUNK
    if not gather:
        assert d == L

    assert ch % 2 == 0

    def body(*refs):
        if gather:
            (t_hbm, row_hbm, col_hbm, ew_hbm, out_hbm,
             rowv, colv, ewv, g_buf, acc, sem0, sem1, ssem0, ssem1) = refs
        else:
            (col_hbm, ew_hbm, out_hbm,
             colv, ewv, g_buf, acc, sem0, sem1, ssem0, ssem1) = refs
            t_hbm = row_hbm = rowv = None
        sems = (sem0, sem1)
        ssems = (ssem0, ssem1)
        cid = lax.axis_index("c")
        sid = lax.axis_index("s")
        wid = sid * NC + cid

        if gather:
            pltpu.sync_copy(row_hbm.at[wid], rowv)
        pltpu.sync_copy(col_hbm.at[wid], colv)
        pltpu.sync_copy(ew_hbm.at[wid], ewv)

        idxs = [jnp.full((L,), j, jnp.int32) for j in range(L)]

        @pl.loop(0, CHUNK)
        def _(r):
            for f in range(d // L):
                g_buf[0, r, pl.ds(f * L, L)] = jnp.zeros((L,), jnp.float32)

        base = sid * rows_per_tile
        for k in range(n_full):
            pltpu.sync_copy(g_buf.at[0], acc.at[pl.ds(base + k * CHUNK, CHUNK)])
        if tail:
            pltpu.sync_copy(g_buf.at[0, pl.ds(0, tail)],
                            acc.at[pl.ds(base + n_full * CHUNK, tail)])

        def issue(ci, b):
            pltpu.async_copy(t_hbm.at[rowv.at[ci]], g_buf.at[b], sems[b])

        def drain(ci, b):
            pltpu.make_async_copy(
                t_hbm.at[rowv.at[ci]], g_buf.at[b], sems[b]).wait()

        def scale(ci, b):
            for g in range(CHUNK // L):
                ew16 = ewv[ci, pl.ds(g * L, L)]
                for j in range(L):
                    r = g * L + j
                    s = ew16.at[idxs[j]].get(mode="promise_in_bounds")
                    if gather:
                        for f in range(d // L):
                            sl = pl.ds(f * L, L)
                            g_buf[b, r, sl] = g_buf[b, r, sl] * s
                    else:
                        sl0 = pl.ds(0, L)
                        g_buf[b, r, sl0] = g_buf[b, r, sl0] * 0.0 + s

        if gather:
            issue(0, 0)
        plsc.subcore_barrier()

        def scat_wait(ci, b):
            pltpu.make_async_copy(
                g_buf.at[b], acc.at[colv.at[ci]], ssems[b]).wait()

        @pl.loop(0, ch, step=2)
        def _(ci):
            for b in range(2):
                cur = ci + b
                nxt = cur + 1

                if gather:
                    @pl.when(nxt < ch)
                    def _():
                        @pl.when(nxt >= 2)
                        def _():
                            scat_wait(nxt - 2, 1 - b)
                        issue(nxt, 1 - b)

                    drain(cur, b)
                else:
                    @pl.when(cur >= 2)
                    def _():
                        scat_wait(cur - 2, b)
                scale(cur, b)
                pltpu.async_copy(g_buf.at[b], acc.at[colv.at[cur]],
                                 ssems[b], add=True)

        for b in range(2):
            scat_wait(ch - 2 + b, b)
        plsc.subcore_barrier()
        pltpu.sync_copy(acc.at[pl.ds(base, rows_per_tile)],
                        out_hbm.at[cid, pl.ds(base, rows_per_tile)])

    scratch = []
    if gather:
        scratch.append(pltpu.VMEM((ch, CHUNK), jnp.int32))
    scratch += [
        pltpu.VMEM((ch, CHUNK), jnp.int32),
        pltpu.VMEM((ch, CHUNK), jnp.float32),
        pltpu.VMEM((2, CHUNK, d), jnp.float32),
        pltpu.VMEM_SHARED((n_nodes, d), jnp.float32),
        pltpu.SemaphoreType.DMA,
        pltpu.SemaphoreType.DMA,
        pltpu.SemaphoreType.DMA,
        pltpu.SemaphoreType.DMA,
    ]
    return pl.kernel(
        body,
        out_type=jax.ShapeDtypeStruct((NC, n_nodes, d), jnp.float32),
        mesh=mesh,
        scratch_types=scratch,
        compiler_params=pltpu.CompilerParams(use_tc_tiling_on_sc=False),
    )


def _tc_matmul1(x, w1):
    n, fin = x.shape
    hid = w1.shape[1]
    bn = 1024

    def body(x_ref, w_ref, hw_ref):
        hw_ref[...] = jnp.dot(
            x_ref[...], w_ref[...], preferred_element_type=jnp.float32)

    return pl.pallas_call(
        body,
        grid=(n // bn,),
        in_specs=[
            pl.BlockSpec((bn, fin), lambda i: (i, 0)),
            pl.BlockSpec((fin, hid), lambda i: (0, 0)),
        ],
        out_specs=pl.BlockSpec((bn, hid), lambda i: (i, 0)),
        out_shape=jax.ShapeDtypeStruct((n, hid), jnp.float32),
    )(x, w1)


def _tc_prescale1(deg3, hw):
    npad = hw.shape[0]
    bn = 1024

    def body(deg_ref, hw_ref, t1_ref, dinv_ref):
        deg = deg_ref[0] + deg_ref[1] + 1.0
        dinv = lax.rsqrt(deg)
        t1_ref[...] = dinv * hw_ref[...]
        dinv_ref[...] = dinv

    return pl.pallas_call(
        body,
        grid=(npad // bn,),
        in_specs=[
            pl.BlockSpec((NC, bn, L), lambda i: (0, i, 0)),
            pl.BlockSpec((bn, L), lambda i: (i, 0)),
        ],
        out_specs=[
            pl.BlockSpec((bn, L), lambda i: (i, 0)),
            pl.BlockSpec((bn, L), lambda i: (i, 0)),
        ],
        out_shape=[
            jax.ShapeDtypeStruct((npad, L), jnp.float32),
            jax.ShapeDtypeStruct((npad, L), jnp.float32),
        ],
    )(deg3, hw)


def _tc_mid(acc3, t1, dinv16, b1r):
    npad = t1.shape[0]
    bn = 1024

    def body(acc_ref, t1_ref, dinv_ref, b1_ref, hid_ref, h2_ref):
        s = acc_ref[0] + acc_ref[1] + t1_ref[...]
        h = jnp.maximum(dinv_ref[...] * s + b1_ref[...], 0.0)
        hid_ref[...] = h
        h2_ref[...] = dinv_ref[...] * h

    return pl.pallas_call(
        body,
        grid=(npad // bn,),
        in_specs=[
            pl.BlockSpec((NC, bn, L), lambda i: (0, i, 0)),
            pl.BlockSpec((bn, L), lambda i: (i, 0)),
            pl.BlockSpec((bn, L), lambda i: (i, 0)),
            pl.BlockSpec((1, L), lambda i: (0, 0)),
        ],
        out_specs=[
            pl.BlockSpec((bn, L), lambda i: (i, 0)),
            pl.BlockSpec((bn, L), lambda i: (i, 0)),
        ],
        out_shape=[
            jax.ShapeDtypeStruct((npad, L), jnp.float32),
            jax.ShapeDtypeStruct((npad, L), jnp.float32),
        ],
    )(acc3, t1, dinv16, b1r)


def _tc_post(acc3, h2, dinv16, w2, b2r):
    npad = h2.shape[0]
    c = b2r.shape[1]
    hid = w2.shape[0]
    bn = 1024

    def body(acc_ref, h2_ref, dinv_ref, w2_ref, b2_ref, out_ref):
        s = acc_ref[0] + acc_ref[1] + h2_ref[...]
        o = dinv_ref[...] * s
        out_ref[...] = jnp.dot(
            o, w2_ref[...], preferred_element_type=jnp.float32) + b2_ref[...]

    return pl.pallas_call(
        body,
        grid=(npad // bn,),
        in_specs=[
            pl.BlockSpec((NC, bn, hid), lambda i: (0, i, 0)),
            pl.BlockSpec((bn, hid), lambda i: (i, 0)),
            pl.BlockSpec((bn, hid), lambda i: (i, 0)),
            pl.BlockSpec((hid, c), lambda i: (0, 0)),
            pl.BlockSpec((1, c), lambda i: (0, 0)),
        ],
        out_specs=pl.BlockSpec((bn, c), lambda i: (i, 0)),
        out_shape=jax.ShapeDtypeStruct((npad, c), jnp.float32),
    )(acc3, h2, dinv16, w2, b2r)


def kernel(x, edge_index, edge_weight, W1, b1, W2, b2):
    n, _ = x.shape
    e = edge_weight.shape[0]
    hid = W1.shape[1]
    npad = ((n + NS * CHUNK - 1) // (NS * CHUNK)) * (NS * CHUNK)

    epw = ((e + 2 * NW * CHUNK - 1) // (2 * NW * CHUNK)) * 2 * CHUNK
    ch = epw // CHUNK
    epad = NW * epw - e
    row3 = jnp.pad(edge_index[0], (0, epad)).reshape(NW, ch, CHUNK)
    col3 = jnp.pad(edge_index[1], (0, epad)).reshape(NW, ch, CHUNK)
    ew3 = jnp.pad(edge_weight, (0, epad)).reshape(NW, ch, CHUNK)
    b1r = b1.reshape(1, L)
    b2r = b2.reshape(1, W2.shape[1])
    xp = jnp.pad(x, ((0, npad - n), (0, 0)))

    hw = _tc_matmul1(xp, W1)
    deg16 = _sc_edge_pass(npad, ch, L, gather=False)(col3, ew3)
    t1, dinv16 = _tc_prescale1(deg16, hw)
    acc1 = _sc_edge_pass(npad, ch, hid, gather=True)(t1, row3, col3, ew3)
    hidden, h2 = _tc_mid(acc1, t1, dinv16, b1r)
    acc2 = _sc_edge_pass(npad, ch, hid, gather=True)(h2, row3, col3, ew3)
    out = _tc_post(acc2, h2, dinv16, W2, b2r)
    return (out[:n], hidden[:n])
